# baseline (speedup 1.0000x reference)
from functools import partial

import jax
import jax.numpy as jnp
from jax.experimental import pallas as pl
from jax.experimental.pallas import tpu as pltpu


def _rup(a, b):
    return (a + b - 1) // b * b


def _pick_tm(m, kp):
    cap = 1200 if kp > 2048 else 1920
    for t in (1920, 1800, 1600, 1440, 1280, 1200, 960, 800, 640, 512):
        if t <= cap and m % t == 0:
            return t
    return min(cap, _rup(m, 8))


def _pick_tn(np_):
    for t in (512, 640):
        if np_ % t == 0:
            return t
    return np_


def _mm_body(x_ref, w_ref, s_ref, b_ref, o_ref, *, act):
    y = jnp.dot(x_ref[...], w_ref[...], preferred_element_type=jnp.float32)
    y = y * s_ref[...] + b_ref[...]
    if act == "relu":
        y = jnp.maximum(y, 0.0)
    o_ref[...] = y.astype(o_ref.dtype)


def _mm(x, w, scale, bias, act=None, out_dtype=jnp.bfloat16, n_pad_to=None):
    m, k = x.shape
    n = w.shape[1]
    kp = _rup(k, 128)
    np_ = n_pad_to if n_pad_to is not None else _rup(n, 128)
    tn = _pick_tn(np_)
    tm = _pick_tm(m, kp)
    mp = _rup(m, tm)

    xb = x.astype(jnp.bfloat16)
    if (mp, kp) != (m, k):
        xb = jnp.pad(xb, ((0, mp - m), (0, kp - k)))
    wb = w.astype(jnp.bfloat16)
    if (kp, np_) != (k, n):
        wb = jnp.pad(wb, ((0, kp - k), (0, np_ - n)))
    sc = scale.astype(jnp.float32).reshape(1, n)
    bi = bias.astype(jnp.float32).reshape(1, n)
    if np_ != n:
        sc = jnp.pad(sc, ((0, 0), (0, np_ - n)))
        bi = jnp.pad(bi, ((0, 0), (0, np_ - n)))

    out = pl.pallas_call(
        partial(_mm_body, act=act),
        out_shape=jax.ShapeDtypeStruct((mp, np_), out_dtype),
        grid=(mp // tm, np_ // tn),
        in_specs=[
            pl.BlockSpec((tm, kp), lambda i, j: (i, 0)),
            pl.BlockSpec((kp, tn), lambda i, j: (0, j)),
            pl.BlockSpec((1, tn), lambda i, j: (0, j)),
            pl.BlockSpec((1, tn), lambda i, j: (0, j)),
        ],
        out_specs=pl.BlockSpec((tm, tn), lambda i, j: (i, j)),
        compiler_params=pltpu.CompilerParams(
            dimension_semantics=("parallel", "parallel"),
            vmem_limit_bytes=58 * 1024 * 1024),
    )(xb, wb, sc, bi)
    return out[:m] if mp != m else out


def _ctx_body(slr_ref, slt_ref, wt_ref, ts_ref, tb_ref, v_ref,
              oi_ref, oo_ref, acc_ref, *, tp, inv_denom):
    q = pl.program_id(2)

    @pl.when(q == 0)
    def _():
        acc_ref[...] = jnp.zeros_like(acc_ref)

    t_pre = jnp.dot(slr_ref[0], wt_ref[...],
                    preferred_element_type=jnp.float32)
    t_sig = jax.nn.sigmoid(-(t_pre * ts_ref[...] + tb_ref[...]))
    s_sig = jax.nn.sigmoid(slt_ref[0].astype(jnp.float32))
    lhs = jnp.concatenate([s_sig, t_sig], axis=0).astype(jnp.bfloat16)
    acc_ref[...] += jnp.dot(lhs, v_ref[0],
                            preferred_element_type=jnp.float32)

    @pl.when(q == pl.num_programs(2) - 1)
    def _():
        out = acc_ref[...] * inv_denom
        oi_ref[0] = out[:tp].astype(oi_ref.dtype)
        oo_ref[0] = out[tp:].astype(oo_ref.dtype)


def _fused_context(sl, wt_p, tsc_p, tbi_p, v_p, hw):
    b, p, qp = sl.shape
    c = v_p.shape[-1]
    tp = 1200
    tq = 512
    inv_denom = 1.0 / float(hw)

    outs = pl.pallas_call(
        partial(_ctx_body, tp=tp, inv_denom=inv_denom),
        out_shape=(jax.ShapeDtypeStruct((b, p, c), jnp.bfloat16),
                   jax.ShapeDtypeStruct((b, p, c), jnp.bfloat16)),
        grid=(b, p // tp, qp // tq),
        in_specs=[
            pl.BlockSpec((1, tp, qp), lambda bb, r, q: (bb, r, 0)),
            pl.BlockSpec((1, tp, tq), lambda bb, r, q: (bb, r, q)),
            pl.BlockSpec((qp, tq), lambda bb, r, q: (0, q)),
            pl.BlockSpec((1, tq), lambda bb, r, q: (0, q)),
            pl.BlockSpec((1, tq), lambda bb, r, q: (0, q)),
            pl.BlockSpec((1, tq, c), lambda bb, r, q: (bb, q, 0)),
        ],
        out_specs=(pl.BlockSpec((1, tp, c), lambda bb, r, q: (bb, r, 0)),
                   pl.BlockSpec((1, tp, c), lambda bb, r, q: (bb, r, 0))),
        scratch_shapes=[pltpu.VMEM((2 * tp, c), jnp.float32)],
        compiler_params=pltpu.CompilerParams(
            dimension_semantics=("parallel", "parallel", "arbitrary"),
            vmem_limit_bytes=58 * 1024 * 1024),
    )(sl, sl, wt_p, tsc_p, tbi_p, v_p)
    return outs


def _head_conv_body(x_hbm, w_hbm, s_ref, b_ref, o_ref,
                    xbuf, wbuf, xsem, wsem, *, th, wp):
    b = pl.program_id(0)
    half = pl.program_id(1)
    cin = xbuf.shape[-1]
    n = o_ref.shape[-1]

    @pl.when(half == 0)
    def _():
        pltpu.make_async_copy(w_hbm, wbuf, wsem).start()

    cpx = pltpu.make_async_copy(
        x_hbm.at[b, pl.ds(half * th, th + 2)], xbuf, xsem)
    cpx.start()

    @pl.when(half == 0)
    def _():
        pltpu.make_async_copy(w_hbm, wbuf, wsem).wait()
    cpx.wait()

    y = None
    for kh in range(3):
        for kw in range(3):
            t = kh * 3 + kw
            lhs = xbuf[kh:kh + th, kw:kw + wp, :].reshape(th * wp, cin)
            d = jnp.dot(lhs, wbuf[t * cin:(t + 1) * cin, :],
                        preferred_element_type=jnp.float32)
            y = d if y is None else y + d

    y = jnp.maximum(y * s_ref[...] + b_ref[...], 0.0)
    o_ref[0] = y.reshape(th, wp, n).astype(o_ref.dtype)


def _head_conv3x3(x, w, scale, bias):
    bsz, h, wdt, cin = x.shape
    n = w.shape[1]
    th = h // 2
    wp = 64
    xp = jnp.pad(x, ((0, 0), (1, 1), (1, wp + 1 - wdt), (0, 0)))
    sc = scale.astype(jnp.float32).reshape(1, n)
    bi = bias.astype(jnp.float32).reshape(1, n)

    return pl.pallas_call(
        partial(_head_conv_body, th=th, wp=wp),
        out_shape=jax.ShapeDtypeStruct((bsz, h, wp, n), jnp.bfloat16),
        grid=(bsz, 2),
        in_specs=[
            pl.BlockSpec(memory_space=pl.ANY),
            pl.BlockSpec(memory_space=pl.ANY),
            pl.BlockSpec((1, n), lambda b, hh: (0, 0)),
            pl.BlockSpec((1, n), lambda b, hh: (0, 0)),
        ],
        out_specs=pl.BlockSpec((1, th, wp, n), lambda b, hh: (b, hh, 0, 0)),
        scratch_shapes=[
            pltpu.VMEM((th + 2, wp + 2, cin), jnp.bfloat16),
            pltpu.VMEM((9 * cin, n), jnp.bfloat16),
            pltpu.SemaphoreType.DMA,
            pltpu.SemaphoreType.DMA,
        ],
        compiler_params=pltpu.CompilerParams(
            dimension_semantics=("parallel", "arbitrary"),
            vmem_limit_bytes=58 * 1024 * 1024),
    )(xp, w.astype(jnp.bfloat16), sc, bi)


def _lsm_body(x_ref, o_ref):
    x = x_ref[...].astype(jnp.float32)
    m = jnp.max(x, axis=1, keepdims=True)
    z = x - m
    lse = jnp.log(jnp.sum(jnp.exp(z), axis=1, keepdims=True))
    o_ref[...] = z - lse


def _log_softmax_nchw(x):
    b, c, h, w = x.shape
    th = 32
    return pl.pallas_call(
        _lsm_body,
        out_shape=jax.ShapeDtypeStruct((b, c, h, w), jnp.float32),
        grid=(b, h // th),
        in_specs=[pl.BlockSpec((1, c, th, w), lambda bb, i: (bb, 0, i, 0))],
        out_specs=pl.BlockSpec((1, c, th, w), lambda bb, i: (bb, 0, i, 0)),
        compiler_params=pltpu.CompilerParams(
            dimension_semantics=("parallel", "parallel"),
            vmem_limit_bytes=40 * 1024 * 1024),
    )(x)


def _cols3x3_s2(x, extra_zero_ch=0):
    b, h, w, c = x.shape
    xp = jnp.pad(x, ((0, 0), (1, 1), (1, 1), (0, 0)))
    ho, wo = h // 2, w // 2
    taps = [xp[:, kh:kh + h:2, kw:kw + w:2, :]
            for kh in range(3) for kw in range(3)]
    if extra_zero_ch:
        taps.append(jnp.zeros((b, ho, wo, extra_zero_ch), x.dtype))
    return jnp.concatenate(taps, axis=-1)


def _conv3x3_s2(x, w, scale, bias, n_pad_to=None, extra_zero_ch=0):
    b, h, wdt, _ = x.shape
    cols = _cols3x3_s2(x, extra_zero_ch=extra_zero_ch)
    k = cols.shape[-1]
    y = _mm(cols.reshape(b * (h // 2) * (wdt // 2), k), w, scale, bias,
            act="relu", n_pad_to=n_pad_to)
    return y.reshape(b, h // 2, wdt // 2, -1)


def _interp_mat(n_in, n_out):
    pos = jnp.arange(n_out, dtype=jnp.float32) * (n_in - 1) / (n_out - 1)
    lo = jnp.clip(jnp.floor(pos).astype(jnp.int32), 0, n_in - 2)
    frac = pos - lo.astype(jnp.float32)
    rows = jnp.arange(n_out)
    mat = jnp.zeros((n_out, n_in), jnp.float32)
    mat = mat.at[rows, lo].add(1.0 - frac)
    mat = mat.at[rows, lo + 1].add(frac)
    return mat


def kernel(data, stem1_w, stem1_scale, stem1_bias, stem2_w, stem2_scale, stem2_bias, layer3_w, layer3_scale, layer3_bias, layer4_w, layer4_scale, layer4_bias, head1_w, head1_scale, head1_bias, head2_w, head2_scale, head2_bias, aux1_w, aux1_scale, aux1_bias, aux2_w, aux2_scale, aux2_bias, context_reduce_w, context_reduce_scale, context_reduce_bias, context_intra1_w, context_intra1_scale, context_intra1_bias, context_intra2_w, context_intra2_scale, context_intra2_bias, context_intra3_w, context_intra3_scale, context_intra3_bias, context_transform_w, context_transform_scale, context_transform_bias, context_intra_post_w, context_intra_post_scale, context_intra_post_bias, context_inter_post_w, context_inter_post_scale, context_inter_post_bias):
    b = data.shape[0]
    x = jnp.transpose(data, (0, 2, 3, 1)).astype(jnp.bfloat16)

    cols1 = _cols3x3_s2(x, extra_zero_ch=128 - 27)
    y = _mm(cols1.reshape(-1, 128), stem1_w, stem1_scale, stem1_bias,
            act="relu", n_pad_to=64)
    c = y.reshape(b, 240, 240, 64)
    w2 = jnp.pad(stem2_w, ((0, 64), (0, 0)))
    c = _conv3x3_s2(c, w2, stem2_scale, stem2_bias, extra_zero_ch=64)
    c3 = _conv3x3_s2(c, layer3_w, layer3_scale, layer3_bias)
    c4 = _mm(c3.reshape(-1, 1024), layer4_w, layer4_scale, layer4_bias,
             act="relu").reshape(b, 60, 60, 2048)

    hw = 3600
    qp = 4096
    v = _mm(c4.reshape(-1, 2048), context_reduce_w, context_reduce_scale,
            context_reduce_bias, act="relu")
    s = _mm(v, context_intra1_w, context_intra1_scale, context_intra1_bias,
            act="relu")
    s = _mm(s, context_intra2_w, context_intra2_scale, context_intra2_bias,
            act="relu", n_pad_to=qp)
    sl = _mm(s, context_intra3_w, context_intra3_scale, context_intra3_bias,
             act=None, n_pad_to=qp)

    wt_p = jnp.pad(context_transform_w.astype(jnp.bfloat16),
                   ((0, qp - hw), (0, qp - hw)))
    tsc_p = jnp.pad(context_transform_scale.astype(jnp.float32).reshape(1, hw),
                    ((0, 0), (0, qp - hw)))
    tbi_p = jnp.pad(context_transform_bias.astype(jnp.float32).reshape(1, hw),
                    ((0, 0), (0, qp - hw)))
    v_p = jnp.pad(v.reshape(b, hw, 512), ((0, 0), (0, qp - hw), (0, 0)))

    intra_ctx, inter_ctx = _fused_context(
        sl.reshape(b, hw, qp), wt_p, tsc_p, tbi_p, v_p, hw)

    intra_ctx = _mm(intra_ctx.reshape(-1, 512), context_intra_post_w,
                    context_intra_post_scale, context_intra_post_bias,
                    act="relu")
    inter_ctx = _mm(inter_ctx.reshape(-1, 512), context_inter_post_w,
                    context_inter_post_scale, context_inter_post_bias,
                    act="relu")

    fm_in = jnp.concatenate(
        [c4, intra_ctx.reshape(b, 60, 60, 512),
         inter_ctx.reshape(b, 60, 60, 512)], axis=-1)

    fm = _head_conv3x3(fm_in, head1_w, head1_scale, head1_bias)
    fm = _mm(fm.reshape(-1, 512), head2_w, head2_scale, head2_bias,
             act=None, out_dtype=jnp.float32)
    fm = fm.reshape(b, 60, 64, 128)[:, :, :60, :19]
    fm = jnp.transpose(fm, (0, 3, 1, 2))

    mh = _interp_mat(60, 480)
    y = jnp.einsum("oh,bchw->bcow", mh, fm)
    y = jnp.einsum("pw,bcow->bcop", mh, y)
    return _log_softmax_nchw(y)


# speedup vs baseline: 1.0166x; 1.0166x over previous
from functools import partial

import jax
import jax.numpy as jnp
from jax.experimental import pallas as pl
from jax.experimental.pallas import tpu as pltpu


def _rup(a, b):
    return (a + b - 1) // b * b


def _pick_tm(m, kp):
    cap = 1200 if kp > 2048 else 1920
    for t in (1920, 1800, 1600, 1440, 1280, 1200, 960, 800, 640, 512):
        if t <= cap and m % t == 0:
            return t
    return min(cap, _rup(m, 8))


def _pick_tn(np_):
    for t in (512, 640):
        if np_ % t == 0:
            return t
    return np_


def _mm_body(x_ref, w_ref, s_ref, b_ref, o_ref, *, act):
    y = jnp.dot(x_ref[...], w_ref[...], preferred_element_type=jnp.float32)
    y = y * s_ref[...] + b_ref[...]
    if act == "relu":
        y = jnp.maximum(y, 0.0)
    o_ref[...] = y.astype(o_ref.dtype)


def _mm(x, w, scale, bias, act=None, out_dtype=jnp.bfloat16, n_pad_to=None):
    m, k = x.shape
    n = w.shape[1]
    kp = _rup(k, 128)
    np_ = n_pad_to if n_pad_to is not None else _rup(n, 128)
    tn = _pick_tn(np_)
    tm = _pick_tm(m, kp)
    mp = _rup(m, tm)

    xb = x.astype(jnp.bfloat16)
    if (mp, kp) != (m, k):
        xb = jnp.pad(xb, ((0, mp - m), (0, kp - k)))
    wb = w.astype(jnp.bfloat16)
    if (kp, np_) != (k, n):
        wb = jnp.pad(wb, ((0, kp - k), (0, np_ - n)))
    sc = scale.astype(jnp.float32).reshape(1, n)
    bi = bias.astype(jnp.float32).reshape(1, n)
    if np_ != n:
        sc = jnp.pad(sc, ((0, 0), (0, np_ - n)))
        bi = jnp.pad(bi, ((0, 0), (0, np_ - n)))

    out = pl.pallas_call(
        partial(_mm_body, act=act),
        out_shape=jax.ShapeDtypeStruct((mp, np_), out_dtype),
        grid=(mp // tm, np_ // tn),
        in_specs=[
            pl.BlockSpec((tm, kp), lambda i, j: (i, 0)),
            pl.BlockSpec((kp, tn), lambda i, j: (0, j)),
            pl.BlockSpec((1, tn), lambda i, j: (0, j)),
            pl.BlockSpec((1, tn), lambda i, j: (0, j)),
        ],
        out_specs=pl.BlockSpec((tm, tn), lambda i, j: (i, j)),
        compiler_params=pltpu.CompilerParams(
            dimension_semantics=("parallel", "parallel"),
            vmem_limit_bytes=58 * 1024 * 1024),
    )(xb, wb, sc, bi)
    return out[:m] if mp != m else out


def _ctx_body(slr_ref, slt_ref, wt_ref, ts_ref, tb_ref, v_ref,
              oi_ref, oo_ref, acc_ref, *, tp, inv_denom):
    q = pl.program_id(2)

    @pl.when(q == 0)
    def _():
        acc_ref[...] = jnp.zeros_like(acc_ref)

    t_pre = jnp.dot(slr_ref[0], wt_ref[...],
                    preferred_element_type=jnp.float32)
    t_sig = jax.nn.sigmoid(-(t_pre * ts_ref[...] + tb_ref[...]))
    s_sig = jax.nn.sigmoid(slt_ref[0].astype(jnp.float32))
    lhs = jnp.concatenate([s_sig, t_sig], axis=0).astype(jnp.bfloat16)
    acc_ref[...] += jnp.dot(lhs, v_ref[0],
                            preferred_element_type=jnp.float32)

    @pl.when(q == pl.num_programs(2) - 1)
    def _():
        out = acc_ref[...] * inv_denom
        oi_ref[0] = out[:tp].astype(oi_ref.dtype)
        oo_ref[0] = out[tp:].astype(oo_ref.dtype)


def _fused_context(sl, wt_p, tsc_p, tbi_p, v_p, hw):
    b, p, qp = sl.shape
    c = v_p.shape[-1]
    tp = 1200
    tq = 512
    inv_denom = 1.0 / float(hw)

    outs = pl.pallas_call(
        partial(_ctx_body, tp=tp, inv_denom=inv_denom),
        out_shape=(jax.ShapeDtypeStruct((b, p, c), jnp.bfloat16),
                   jax.ShapeDtypeStruct((b, p, c), jnp.bfloat16)),
        grid=(b, p // tp, qp // tq),
        in_specs=[
            pl.BlockSpec((1, tp, qp), lambda bb, r, q: (bb, r, 0)),
            pl.BlockSpec((1, tp, tq), lambda bb, r, q: (bb, r, q)),
            pl.BlockSpec((qp, tq), lambda bb, r, q: (0, q)),
            pl.BlockSpec((1, tq), lambda bb, r, q: (0, q)),
            pl.BlockSpec((1, tq), lambda bb, r, q: (0, q)),
            pl.BlockSpec((1, tq, c), lambda bb, r, q: (bb, q, 0)),
        ],
        out_specs=(pl.BlockSpec((1, tp, c), lambda bb, r, q: (bb, r, 0)),
                   pl.BlockSpec((1, tp, c), lambda bb, r, q: (bb, r, 0))),
        scratch_shapes=[pltpu.VMEM((2 * tp, c), jnp.float32)],
        compiler_params=pltpu.CompilerParams(
            dimension_semantics=("parallel", "parallel", "arbitrary"),
            vmem_limit_bytes=58 * 1024 * 1024),
    )(sl, sl, wt_p, tsc_p, tbi_p, v_p)
    return outs


def _head_conv_body(x_hbm, w_ref, s_ref, b_ref, o_ref,
                    xbuf, acc_ref, xsem, *, th, wp):
    b = pl.program_id(0)
    half = pl.program_id(1)
    kh = pl.program_id(2)
    cin = xbuf.shape[-1]
    n = o_ref.shape[-1]

    @pl.when(kh == 0)
    def _():
        cp = pltpu.make_async_copy(
            x_hbm.at[b, pl.ds(half * th, th + 2)], xbuf, xsem)
        cp.start()
        cp.wait()

    y = None
    for kw in range(3):
        lhs = xbuf[pl.ds(kh, th), kw:kw + wp, :].reshape(th * wp, cin)
        d = jnp.dot(lhs, w_ref[0, kw * cin:(kw + 1) * cin, :],
                    preferred_element_type=jnp.float32)
        y = d if y is None else y + d

    @pl.when(kh == 0)
    def _():
        acc_ref[...] = y

    @pl.when(kh > 0)
    def _():
        acc_ref[...] += y

    @pl.when(kh == 2)
    def _():
        z = jnp.maximum(acc_ref[...] * s_ref[...] + b_ref[...], 0.0)
        o_ref[0] = z.reshape(th, wp, n).astype(o_ref.dtype)


def _head_conv3x3(x, w, scale, bias):
    bsz, h, wdt, cin = x.shape
    n = w.shape[1]
    th = h // 2
    wp = 64
    xp = jnp.pad(x, ((0, 0), (1, 1), (1, wp + 1 - wdt), (0, 0)))
    sc = scale.astype(jnp.float32).reshape(1, n)
    bi = bias.astype(jnp.float32).reshape(1, n)

    wr = w.astype(jnp.bfloat16).reshape(3, 3 * cin, n)
    return pl.pallas_call(
        partial(_head_conv_body, th=th, wp=wp),
        out_shape=jax.ShapeDtypeStruct((bsz, h, wp, n), jnp.bfloat16),
        grid=(bsz, 2, 3),
        in_specs=[
            pl.BlockSpec(memory_space=pl.ANY),
            pl.BlockSpec((1, 3 * cin, n), lambda b, hh, kh: (kh, 0, 0)),
            pl.BlockSpec((1, n), lambda b, hh, kh: (0, 0)),
            pl.BlockSpec((1, n), lambda b, hh, kh: (0, 0)),
        ],
        out_specs=pl.BlockSpec((1, th, wp, n),
                               lambda b, hh, kh: (b, hh, 0, 0)),
        scratch_shapes=[
            pltpu.VMEM((th + 2, wp + 2, cin), jnp.bfloat16),
            pltpu.VMEM((th * wp, n), jnp.float32),
            pltpu.SemaphoreType.DMA,
        ],
        compiler_params=pltpu.CompilerParams(
            dimension_semantics=("parallel", "parallel", "arbitrary"),
            vmem_limit_bytes=58 * 1024 * 1024),
    )(xp, wr, sc, bi)


def _lsm_body(x_ref, o_ref):
    x = x_ref[...].astype(jnp.float32)
    m = jnp.max(x, axis=1, keepdims=True)
    z = x - m
    lse = jnp.log(jnp.sum(jnp.exp(z), axis=1, keepdims=True))
    o_ref[...] = z - lse


def _log_softmax_nchw(x):
    b, c, h, w = x.shape
    th = 32
    return pl.pallas_call(
        _lsm_body,
        out_shape=jax.ShapeDtypeStruct((b, c, h, w), jnp.float32),
        grid=(b, h // th),
        in_specs=[pl.BlockSpec((1, c, th, w), lambda bb, i: (bb, 0, i, 0))],
        out_specs=pl.BlockSpec((1, c, th, w), lambda bb, i: (bb, 0, i, 0)),
        compiler_params=pltpu.CompilerParams(
            dimension_semantics=("parallel", "parallel"),
            vmem_limit_bytes=40 * 1024 * 1024),
    )(x)


def _cols3x3_s2(x, extra_zero_ch=0):
    b, h, w, c = x.shape
    xp = jnp.pad(x, ((0, 0), (1, 1), (1, 1), (0, 0)))
    ho, wo = h // 2, w // 2
    taps = [xp[:, kh:kh + h:2, kw:kw + w:2, :]
            for kh in range(3) for kw in range(3)]
    if extra_zero_ch:
        taps.append(jnp.zeros((b, ho, wo, extra_zero_ch), x.dtype))
    return jnp.concatenate(taps, axis=-1)


def _conv3x3_s2(x, w, scale, bias, n_pad_to=None, extra_zero_ch=0):
    b, h, wdt, _ = x.shape
    cols = _cols3x3_s2(x, extra_zero_ch=extra_zero_ch)
    k = cols.shape[-1]
    y = _mm(cols.reshape(b * (h // 2) * (wdt // 2), k), w, scale, bias,
            act="relu", n_pad_to=n_pad_to)
    return y.reshape(b, h // 2, wdt // 2, -1)


def _interp_mat(n_in, n_out):
    pos = jnp.arange(n_out, dtype=jnp.float32) * (n_in - 1) / (n_out - 1)
    lo = jnp.clip(jnp.floor(pos).astype(jnp.int32), 0, n_in - 2)
    frac = pos - lo.astype(jnp.float32)
    rows = jnp.arange(n_out)
    mat = jnp.zeros((n_out, n_in), jnp.float32)
    mat = mat.at[rows, lo].add(1.0 - frac)
    mat = mat.at[rows, lo + 1].add(frac)
    return mat


def kernel(data, stem1_w, stem1_scale, stem1_bias, stem2_w, stem2_scale, stem2_bias, layer3_w, layer3_scale, layer3_bias, layer4_w, layer4_scale, layer4_bias, head1_w, head1_scale, head1_bias, head2_w, head2_scale, head2_bias, aux1_w, aux1_scale, aux1_bias, aux2_w, aux2_scale, aux2_bias, context_reduce_w, context_reduce_scale, context_reduce_bias, context_intra1_w, context_intra1_scale, context_intra1_bias, context_intra2_w, context_intra2_scale, context_intra2_bias, context_intra3_w, context_intra3_scale, context_intra3_bias, context_transform_w, context_transform_scale, context_transform_bias, context_intra_post_w, context_intra_post_scale, context_intra_post_bias, context_inter_post_w, context_inter_post_scale, context_inter_post_bias):
    b = data.shape[0]
    x = jnp.transpose(data, (0, 2, 3, 1)).astype(jnp.bfloat16)

    cols1 = _cols3x3_s2(x, extra_zero_ch=128 - 27)
    y = _mm(cols1.reshape(-1, 128), stem1_w, stem1_scale, stem1_bias,
            act="relu", n_pad_to=64)
    c = y.reshape(b, 240, 240, 64)
    w2 = jnp.pad(stem2_w, ((0, 64), (0, 0)))
    c = _conv3x3_s2(c, w2, stem2_scale, stem2_bias, extra_zero_ch=64)
    c3 = _conv3x3_s2(c, layer3_w, layer3_scale, layer3_bias)
    c4 = _mm(c3.reshape(-1, 1024), layer4_w, layer4_scale, layer4_bias,
             act="relu").reshape(b, 60, 60, 2048)

    hw = 3600
    qp = 4096
    v = _mm(c4.reshape(-1, 2048), context_reduce_w, context_reduce_scale,
            context_reduce_bias, act="relu")
    s = _mm(v, context_intra1_w, context_intra1_scale, context_intra1_bias,
            act="relu")
    s = _mm(s, context_intra2_w, context_intra2_scale, context_intra2_bias,
            act="relu", n_pad_to=qp)
    sl = _mm(s, context_intra3_w, context_intra3_scale, context_intra3_bias,
             act=None, n_pad_to=qp)

    wt_p = jnp.pad(context_transform_w.astype(jnp.bfloat16),
                   ((0, qp - hw), (0, qp - hw)))
    tsc_p = jnp.pad(context_transform_scale.astype(jnp.float32).reshape(1, hw),
                    ((0, 0), (0, qp - hw)))
    tbi_p = jnp.pad(context_transform_bias.astype(jnp.float32).reshape(1, hw),
                    ((0, 0), (0, qp - hw)))
    v_p = jnp.pad(v.reshape(b, hw, 512), ((0, 0), (0, qp - hw), (0, 0)))

    intra_ctx, inter_ctx = _fused_context(
        sl.reshape(b, hw, qp), wt_p, tsc_p, tbi_p, v_p, hw)

    intra_ctx = _mm(intra_ctx.reshape(-1, 512), context_intra_post_w,
                    context_intra_post_scale, context_intra_post_bias,
                    act="relu")
    inter_ctx = _mm(inter_ctx.reshape(-1, 512), context_inter_post_w,
                    context_inter_post_scale, context_inter_post_bias,
                    act="relu")

    fm_in = jnp.concatenate(
        [c4, intra_ctx.reshape(b, 60, 60, 512),
         inter_ctx.reshape(b, 60, 60, 512)], axis=-1)

    fm = _head_conv3x3(fm_in, head1_w, head1_scale, head1_bias)
    fm = _mm(fm.reshape(-1, 512), head2_w, head2_scale, head2_bias,
             act=None, out_dtype=jnp.float32)
    fm = fm.reshape(b, 60, 64, 128)[:, :, :60, :19]
    fm = jnp.transpose(fm, (0, 3, 1, 2))

    mh = _interp_mat(60, 480)
    y = jnp.einsum("oh,bchw->bcow", mh, fm)
    y = jnp.einsum("pw,bcow->bcop", mh, y)
    return _log_softmax_nchw(y)


# speedup vs baseline: 1.0714x; 1.0539x over previous
from functools import partial

import jax
import jax.numpy as jnp
from jax.experimental import pallas as pl
from jax.experimental.pallas import tpu as pltpu


def _rup(a, b):
    return (a + b - 1) // b * b


def _pick_tm(m, kp):
    cap = 1200 if kp > 2048 else 1920
    for t in (1920, 1800, 1600, 1440, 1280, 1200, 960, 800, 640, 512):
        if t <= cap and m % t == 0:
            return t
    return min(cap, _rup(m, 8))


def _pick_tn(np_):
    for t in (512, 640):
        if np_ % t == 0:
            return t
    return np_


def _mm_body(x_ref, w_ref, s_ref, b_ref, o_ref, *, act):
    y = jnp.dot(x_ref[...], w_ref[...], preferred_element_type=jnp.float32)
    y = y * s_ref[...] + b_ref[...]
    if act == "relu":
        y = jnp.maximum(y, 0.0)
    o_ref[...] = y.astype(o_ref.dtype)


def _mm(x, w, scale, bias, act=None, out_dtype=jnp.bfloat16, n_pad_to=None):
    m, k = x.shape
    n = w.shape[1]
    kp = _rup(k, 128)
    np_ = n_pad_to if n_pad_to is not None else _rup(n, 128)
    tn = _pick_tn(np_)
    tm = _pick_tm(m, kp)
    mp = _rup(m, tm)

    xb = x.astype(jnp.bfloat16)
    if (mp, kp) != (m, k):
        xb = jnp.pad(xb, ((0, mp - m), (0, kp - k)))
    wb = w.astype(jnp.bfloat16)
    if (kp, np_) != (k, n):
        wb = jnp.pad(wb, ((0, kp - k), (0, np_ - n)))
    sc = scale.astype(jnp.float32).reshape(1, n)
    bi = bias.astype(jnp.float32).reshape(1, n)
    if np_ != n:
        sc = jnp.pad(sc, ((0, 0), (0, np_ - n)))
        bi = jnp.pad(bi, ((0, 0), (0, np_ - n)))

    out = pl.pallas_call(
        partial(_mm_body, act=act),
        out_shape=jax.ShapeDtypeStruct((mp, np_), out_dtype),
        grid=(mp // tm, np_ // tn),
        in_specs=[
            pl.BlockSpec((tm, kp), lambda i, j: (i, 0)),
            pl.BlockSpec((kp, tn), lambda i, j: (0, j)),
            pl.BlockSpec((1, tn), lambda i, j: (0, j)),
            pl.BlockSpec((1, tn), lambda i, j: (0, j)),
        ],
        out_specs=pl.BlockSpec((tm, tn), lambda i, j: (i, j)),
        compiler_params=pltpu.CompilerParams(
            dimension_semantics=("parallel", "parallel"),
            vmem_limit_bytes=58 * 1024 * 1024),
    )(xb, wb, sc, bi)
    return out[:m] if mp != m else out


def _mmt_body(x_ref, w_ref, s_ref, b_ref, o_ref):
    y = jnp.dot(x_ref[...], w_ref[...], preferred_element_type=jnp.float32)
    y = jnp.maximum(y * s_ref[...] + b_ref[...], 0.0)
    o_ref[...] = y.astype(o_ref.dtype)


def _mm_rowaffine(x, w, scale, bias, tn=512):
    m, k = x.shape
    n = w.shape[1]
    kp = _rup(k, 128)
    xb = x.astype(jnp.bfloat16)
    if kp != k:
        xb = jnp.pad(xb, ((0, 0), (0, kp - k)))
    sc = scale.astype(jnp.float32).reshape(m, 1)
    bi = bias.astype(jnp.float32).reshape(m, 1)
    return pl.pallas_call(
        _mmt_body,
        out_shape=jax.ShapeDtypeStruct((m, n), jnp.bfloat16),
        grid=(n // tn,),
        in_specs=[
            pl.BlockSpec((m, kp), lambda j: (0, 0)),
            pl.BlockSpec((kp, tn), lambda j: (0, j)),
            pl.BlockSpec((m, 1), lambda j: (0, 0)),
            pl.BlockSpec((m, 1), lambda j: (0, 0)),
        ],
        out_specs=pl.BlockSpec((m, tn), lambda j: (0, j)),
        compiler_params=pltpu.CompilerParams(
            dimension_semantics=("parallel",),
            vmem_limit_bytes=40 * 1024 * 1024),
    )(xb, w.astype(jnp.bfloat16), sc, bi)


def _stem1_transposed(data, w, scale, bias):
    bsz = data.shape[0]
    xp = jnp.pad(data, ((0, 0), (0, 0), (1, 1), (1, 1)))
    xr = xp.reshape(bsz, 3, 241, 2, 241, 2)
    slabs = []
    for kh in range(3):
        for kw in range(3):
            pp = xr[:, :, :, kh % 2, :, kw % 2]
            slabs.append(pp[:, :, kh // 2:kh // 2 + 240, kw // 2:kw // 2 + 240])
    colsT = jnp.concatenate(slabs, axis=1)
    colsT = colsT.transpose(1, 0, 2, 3).reshape(27, -1)
    colsT = jnp.pad(colsT.astype(jnp.bfloat16), ((0, 101), (0, 0)))
    outT = _mm_rowaffine(w.astype(jnp.float32).T, colsT, scale, bias)
    return outT.reshape(64, bsz, 240, 240).transpose(1, 2, 3, 0)


def _ctx_body(slr_ref, slt_ref, wt_ref, ts_ref, tb_ref, v_ref,
              oi_ref, oo_ref, acc_ref, *, tp, inv_denom):
    q = pl.program_id(2)

    @pl.when(q == 0)
    def _():
        acc_ref[...] = jnp.zeros_like(acc_ref)

    t_pre = jnp.dot(slr_ref[0], wt_ref[...],
                    preferred_element_type=jnp.float32)
    t_sig = jax.nn.sigmoid(-(t_pre * ts_ref[...] + tb_ref[...]))
    s_sig = jax.nn.sigmoid(slt_ref[0].astype(jnp.float32))
    lhs = jnp.concatenate([s_sig, t_sig], axis=0).astype(jnp.bfloat16)
    acc_ref[...] += jnp.dot(lhs, v_ref[0],
                            preferred_element_type=jnp.float32)

    @pl.when(q == pl.num_programs(2) - 1)
    def _():
        out = acc_ref[...] * inv_denom
        oi_ref[0] = out[:tp].astype(oi_ref.dtype)
        oo_ref[0] = out[tp:].astype(oo_ref.dtype)


def _fused_context(sl, wt_p, tsc_p, tbi_p, v_p, hw):
    b, p, qp = sl.shape
    c = v_p.shape[-1]
    tp = 1200
    tq = 512
    inv_denom = 1.0 / float(hw)

    outs = pl.pallas_call(
        partial(_ctx_body, tp=tp, inv_denom=inv_denom),
        out_shape=(jax.ShapeDtypeStruct((b, p, c), jnp.bfloat16),
                   jax.ShapeDtypeStruct((b, p, c), jnp.bfloat16)),
        grid=(b, p // tp, qp // tq),
        in_specs=[
            pl.BlockSpec((1, tp, qp), lambda bb, r, q: (bb, r, 0)),
            pl.BlockSpec((1, tp, tq), lambda bb, r, q: (bb, r, q)),
            pl.BlockSpec((qp, tq), lambda bb, r, q: (0, q)),
            pl.BlockSpec((1, tq), lambda bb, r, q: (0, q)),
            pl.BlockSpec((1, tq), lambda bb, r, q: (0, q)),
            pl.BlockSpec((1, tq, c), lambda bb, r, q: (bb, q, 0)),
        ],
        out_specs=(pl.BlockSpec((1, tp, c), lambda bb, r, q: (bb, r, 0)),
                   pl.BlockSpec((1, tp, c), lambda bb, r, q: (bb, r, 0))),
        scratch_shapes=[pltpu.VMEM((2 * tp, c), jnp.float32)],
        compiler_params=pltpu.CompilerParams(
            dimension_semantics=("parallel", "parallel", "arbitrary"),
            vmem_limit_bytes=58 * 1024 * 1024),
    )(sl, sl, wt_p, tsc_p, tbi_p, v_p)
    return outs


def _head_conv_body(x_hbm, w_ref, s_ref, b_ref, o_ref,
                    xbuf, acc_ref, xsem, *, th, wp):
    b = pl.program_id(0)
    half = pl.program_id(1)
    kh = pl.program_id(2)
    cin = xbuf.shape[-1]
    n = o_ref.shape[-1]

    @pl.when(kh == 0)
    def _():
        cp = pltpu.make_async_copy(
            x_hbm.at[b, pl.ds(half * th, th + 2)], xbuf, xsem)
        cp.start()
        cp.wait()

    y = None
    for kw in range(3):
        lhs = xbuf[pl.ds(kh, th), kw:kw + wp, :].reshape(th * wp, cin)
        d = jnp.dot(lhs, w_ref[0, kw * cin:(kw + 1) * cin, :],
                    preferred_element_type=jnp.float32)
        y = d if y is None else y + d

    @pl.when(kh == 0)
    def _():
        acc_ref[...] = y

    @pl.when(kh > 0)
    def _():
        acc_ref[...] += y

    @pl.when(kh == 2)
    def _():
        z = jnp.maximum(acc_ref[...] * s_ref[...] + b_ref[...], 0.0)
        o_ref[0] = z.reshape(th, wp, n).astype(o_ref.dtype)


def _head_conv3x3(x, w, scale, bias):
    bsz, h, wdt, cin = x.shape
    n = w.shape[1]
    th = h // 2
    wp = 64
    xp = jnp.pad(x, ((0, 0), (1, 1), (1, wp + 1 - wdt), (0, 0)))
    sc = scale.astype(jnp.float32).reshape(1, n)
    bi = bias.astype(jnp.float32).reshape(1, n)

    wr = w.astype(jnp.bfloat16).reshape(3, 3 * cin, n)
    return pl.pallas_call(
        partial(_head_conv_body, th=th, wp=wp),
        out_shape=jax.ShapeDtypeStruct((bsz, h, wp, n), jnp.bfloat16),
        grid=(bsz, 2, 3),
        in_specs=[
            pl.BlockSpec(memory_space=pl.ANY),
            pl.BlockSpec((1, 3 * cin, n), lambda b, hh, kh: (kh, 0, 0)),
            pl.BlockSpec((1, n), lambda b, hh, kh: (0, 0)),
            pl.BlockSpec((1, n), lambda b, hh, kh: (0, 0)),
        ],
        out_specs=pl.BlockSpec((1, th, wp, n),
                               lambda b, hh, kh: (b, hh, 0, 0)),
        scratch_shapes=[
            pltpu.VMEM((th + 2, wp + 2, cin), jnp.bfloat16),
            pltpu.VMEM((th * wp, n), jnp.float32),
            pltpu.SemaphoreType.DMA,
        ],
        compiler_params=pltpu.CompilerParams(
            dimension_semantics=("parallel", "parallel", "arbitrary"),
            vmem_limit_bytes=58 * 1024 * 1024),
    )(xp, wr, sc, bi)


def _lsm_body(x_ref, o_ref):
    x = x_ref[...].astype(jnp.float32)
    m = jnp.max(x, axis=1, keepdims=True)
    z = x - m
    lse = jnp.log(jnp.sum(jnp.exp(z), axis=1, keepdims=True))
    o_ref[...] = z - lse


def _log_softmax_nchw(x):
    b, c, h, w = x.shape
    th = 32
    return pl.pallas_call(
        _lsm_body,
        out_shape=jax.ShapeDtypeStruct((b, c, h, w), jnp.float32),
        grid=(b, h // th),
        in_specs=[pl.BlockSpec((1, c, th, w), lambda bb, i: (bb, 0, i, 0))],
        out_specs=pl.BlockSpec((1, c, th, w), lambda bb, i: (bb, 0, i, 0)),
        compiler_params=pltpu.CompilerParams(
            dimension_semantics=("parallel", "parallel"),
            vmem_limit_bytes=40 * 1024 * 1024),
    )(x)


def _cols3x3_s2(x, extra_zero_ch=0):
    b, h, w, c = x.shape
    xp = jnp.pad(x, ((0, 0), (1, 1), (1, 1), (0, 0)))
    ho, wo = h // 2, w // 2
    taps = [xp[:, kh:kh + h:2, kw:kw + w:2, :]
            for kh in range(3) for kw in range(3)]
    if extra_zero_ch:
        taps.append(jnp.zeros((b, ho, wo, extra_zero_ch), x.dtype))
    return jnp.concatenate(taps, axis=-1)


def _conv3x3_s2(x, w, scale, bias, n_pad_to=None, extra_zero_ch=0):
    b, h, wdt, _ = x.shape
    cols = _cols3x3_s2(x, extra_zero_ch=extra_zero_ch)
    k = cols.shape[-1]
    y = _mm(cols.reshape(b * (h // 2) * (wdt // 2), k), w, scale, bias,
            act="relu", n_pad_to=n_pad_to)
    return y.reshape(b, h // 2, wdt // 2, -1)


def _interp_mat(n_in, n_out):
    pos = jnp.arange(n_out, dtype=jnp.float32) * (n_in - 1) / (n_out - 1)
    lo = jnp.clip(jnp.floor(pos).astype(jnp.int32), 0, n_in - 2)
    frac = pos - lo.astype(jnp.float32)
    rows = jnp.arange(n_out)
    mat = jnp.zeros((n_out, n_in), jnp.float32)
    mat = mat.at[rows, lo].add(1.0 - frac)
    mat = mat.at[rows, lo + 1].add(frac)
    return mat


def kernel(data, stem1_w, stem1_scale, stem1_bias, stem2_w, stem2_scale, stem2_bias, layer3_w, layer3_scale, layer3_bias, layer4_w, layer4_scale, layer4_bias, head1_w, head1_scale, head1_bias, head2_w, head2_scale, head2_bias, aux1_w, aux1_scale, aux1_bias, aux2_w, aux2_scale, aux2_bias, context_reduce_w, context_reduce_scale, context_reduce_bias, context_intra1_w, context_intra1_scale, context_intra1_bias, context_intra2_w, context_intra2_scale, context_intra2_bias, context_intra3_w, context_intra3_scale, context_intra3_bias, context_transform_w, context_transform_scale, context_transform_bias, context_intra_post_w, context_intra_post_scale, context_intra_post_bias, context_inter_post_w, context_inter_post_scale, context_inter_post_bias):
    b = data.shape[0]

    c = _stem1_transposed(data, stem1_w, stem1_scale, stem1_bias)
    w2 = jnp.pad(stem2_w, ((0, 64), (0, 0)))
    c = _conv3x3_s2(c, w2, stem2_scale, stem2_bias, extra_zero_ch=64)
    c3 = _conv3x3_s2(c, layer3_w, layer3_scale, layer3_bias)
    c4 = _mm(c3.reshape(-1, 1024), layer4_w, layer4_scale, layer4_bias,
             act="relu").reshape(b, 60, 60, 2048)

    hw = 3600
    qp = 4096
    v = _mm(c4.reshape(-1, 2048), context_reduce_w, context_reduce_scale,
            context_reduce_bias, act="relu")
    s = _mm(v, context_intra1_w, context_intra1_scale, context_intra1_bias,
            act="relu")
    s = _mm(s, context_intra2_w, context_intra2_scale, context_intra2_bias,
            act="relu", n_pad_to=qp)
    sl = _mm(s, context_intra3_w, context_intra3_scale, context_intra3_bias,
             act=None, n_pad_to=qp)

    wt_p = jnp.pad(context_transform_w.astype(jnp.bfloat16),
                   ((0, qp - hw), (0, qp - hw)))
    tsc_p = jnp.pad(context_transform_scale.astype(jnp.float32).reshape(1, hw),
                    ((0, 0), (0, qp - hw)))
    tbi_p = jnp.pad(context_transform_bias.astype(jnp.float32).reshape(1, hw),
                    ((0, 0), (0, qp - hw)))
    v_p = jnp.pad(v.reshape(b, hw, 512), ((0, 0), (0, qp - hw), (0, 0)))

    intra_ctx, inter_ctx = _fused_context(
        sl.reshape(b, hw, qp), wt_p, tsc_p, tbi_p, v_p, hw)

    intra_ctx = _mm(intra_ctx.reshape(-1, 512), context_intra_post_w,
                    context_intra_post_scale, context_intra_post_bias,
                    act="relu")
    inter_ctx = _mm(inter_ctx.reshape(-1, 512), context_inter_post_w,
                    context_inter_post_scale, context_inter_post_bias,
                    act="relu")

    fm_in = jnp.concatenate(
        [c4, intra_ctx.reshape(b, 60, 60, 512),
         inter_ctx.reshape(b, 60, 60, 512)], axis=-1)

    fm = _head_conv3x3(fm_in, head1_w, head1_scale, head1_bias)
    fm = _mm(fm.reshape(-1, 512), head2_w, head2_scale, head2_bias,
             act=None, out_dtype=jnp.float32)
    fm = fm.reshape(b, 60, 64, 128)[:, :, :60, :19]
    fm = jnp.transpose(fm, (0, 3, 1, 2))

    mh = _interp_mat(60, 480)
    y = jnp.einsum("oh,bchw->bcow", mh, fm)
    y = jnp.einsum("pw,bcow->bcop", mh, y)
    return _log_softmax_nchw(y)
